# baseline (speedup 1.0000x reference)
import functools
import math

import jax
import jax.numpy as jnp
from jax import lax
from jax.experimental import pallas as pl
from jax.experimental.pallas import tpu as pltpu
from jax.experimental.pallas import tpu_sc as plsc

N = 4096
E = 65536
RAW = 16
H = 128
OUT = 128
NEXP = 8
NGRAPH = 8

NC = 2
NS = 16
NW = NC * NS
EW = E // NW
CH = 128
NCH = EW // CH
RB = 512


def _enc_body(x_ref, w_ref, b_ref, o_ref):
    h = jnp.dot(x_ref[...], w_ref[...], preferred_element_type=jnp.float32)
    o_ref[...] = jnp.maximum(h + b_ref[...], 0.0)


def _encoder(x, w_pad, b):
    return pl.pallas_call(
        _enc_body,
        out_shape=jax.ShapeDtypeStruct((N, H), jnp.float32),
    )(x, w_pad, b)


def _sc_agg_body(h_hbm, src_hbm, dst_hbm, zer_hbm, out_hbm,
                 src_v, dst_v, rows0_v, rows1_v, rows2_v,
                 acc_sh, sg0, sg1, sg2, ss0, ss1, ss2):
    c = lax.axis_index("c")
    s = lax.axis_index("s")
    w = c * NS + s
    rpt = N // NS
    for r in range(rpt // CH):
        pltpu.sync_copy(zer_hbm, acc_sh.at[pl.ds(s * rpt + r * CH, CH)])
    pltpu.sync_copy(src_hbm.at[w], src_v)
    pltpu.sync_copy(dst_hbm.at[w], dst_v)
    plsc.subcore_barrier()
    rows = (rows0_v, rows1_v, rows2_v)
    sg = (sg0, sg1, sg2)
    ss = (ss0, ss1, ss2)
    nb = 3
    g = [None] * NCH
    sc = [None] * NCH
    for k in range(nb):
        g[k] = pltpu.async_copy(h_hbm.at[src_v.at[k]], rows[k], sg[k])
    for k in range(NCH):
        b = k % nb
        g[k].wait()
        sc[k] = pltpu.async_copy(rows[b], acc_sh.at[dst_v.at[k]], ss[b], add=True)
        if k + nb < NCH:
            sc[k].wait()
            g[k + nb] = pltpu.async_copy(h_hbm.at[src_v.at[k + nb]], rows[b], sg[b])
    for k in range(NCH - nb, NCH):
        sc[k].wait()
    plsc.subcore_barrier()
    pltpu.sync_copy(acc_sh.at[pl.ds(s * rpt, rpt)],
                    out_hbm.at[pl.ds(c * N + s * rpt, rpt)])


def _sc_agg(h, src, dst):
    zer = jnp.zeros((CH, H), jnp.float32)
    src3 = src.reshape(NW, NCH, CH)
    dst3 = dst.reshape(NW, NCH, CH)
    mesh = plsc.VectorSubcoreMesh(core_axis_name="c", subcore_axis_name="s")
    out = pl.kernel(
        _sc_agg_body,
        out_type=jax.ShapeDtypeStruct((NC * N, H), jnp.float32),
        mesh=mesh,
        scratch_types=[
            pltpu.VMEM((NCH, CH), jnp.int32),
            pltpu.VMEM((NCH, CH), jnp.int32),
            pltpu.VMEM((CH, H), jnp.float32),
            pltpu.VMEM((CH, H), jnp.float32),
            pltpu.VMEM((CH, H), jnp.float32),
            pltpu.VMEM_SHARED((N, H), jnp.float32),
            pltpu.SemaphoreType.DMA,
            pltpu.SemaphoreType.DMA,
            pltpu.SemaphoreType.DMA,
            pltpu.SemaphoreType.DMA,
            pltpu.SemaphoreType.DMA,
            pltpu.SemaphoreType.DMA,
        ],
    )(h, src3, dst3, zer)
    return out.reshape(NC, N, H)


NHI = N // CH


def _sizefeat_body(dst2_ref, batch2_ref, sf1_ref, sf2_ref):
    hi_iota = lax.broadcasted_iota(jnp.int32, (NHI, CH), 0)
    lo_iota = lax.broadcasted_iota(jnp.int32, (CH, CH), 1)

    def brow(g, acc):
        row = dst2_ref[pl.ds(g, 1), :]
        hi = row // CH
        lo_t = jnp.transpose(row % CH)
        a = (hi == hi_iota).astype(jnp.float32)
        b = (lo_t == lo_iota).astype(jnp.float32)
        return acc + jnp.dot(a, b, preferred_element_type=jnp.float32)

    deg = lax.fori_loop(0, EB, brow, jnp.zeros((NHI, CH), jnp.float32))
    b2 = batch2_ref[...]
    gs = jnp.zeros((NHI, CH), jnp.float32)
    for g in range(NGRAPH):
        m = (b2 == g)
        cnt = jnp.sum(m.astype(jnp.float32))
        gs = gs + jnp.where(m, cnt, 0.0)
    sf1_ref[...] = jnp.log(1.0 + gs)
    sf2_ref[...] = jnp.log(1.0 + deg)


def _sizefeat(dst, batch):
    return pl.pallas_call(
        _sizefeat_body,
        out_shape=[jax.ShapeDtypeStruct((NHI, CH), jnp.float32)] * 2,
    )(dst.reshape(EB, CH), batch.reshape(NHI, CH))


def _router_body(h_ref, sf1_ref, sf2_ref,
                 wqh_ref, wqs_ref, bq_ref,
                 wkh_ref, wks_ref, bk_ref,
                 wvh_ref, wvs_ref, bv_ref,
                 wo_ref, bo_ref,
                 e2_ref, g2_ref,
                 k_s, v_s):
    i = pl.program_id(0)

    @pl.when(i == 0)
    def _init():
        sf1 = sf1_ref[...]
        sf2 = sf2_ref[...]
        h = h_ref[...]
        k_s[...] = (jnp.dot(h, wkh_ref[...], preferred_element_type=jnp.float32)
                    + sf1 * wks_ref[0:1, :] + sf2 * wks_ref[1:2, :] + bk_ref[...])
        v_s[...] = (jnp.dot(h, wvh_ref[...], preferred_element_type=jnp.float32)
                    + sf1 * wvs_ref[0:1, :] + sf2 * wvs_ref[1:2, :] + bv_ref[...])

    rows = pl.ds(i * RB, RB)
    hq = h_ref[rows, :]
    q = (jnp.dot(hq, wqh_ref[...], preferred_element_type=jnp.float32)
         + sf1_ref[rows, :] * wqs_ref[0:1, :]
         + sf2_ref[rows, :] * wqs_ref[1:2, :] + bq_ref[...])
    scores = lax.dot_general(q, k_s[...], (((1,), (1,)), ((), ())),
                             preferred_element_type=jnp.float32)
    ex = jnp.exp(scores / jnp.sqrt(jnp.float32(H + 2)))
    ssum = jnp.sum(ex, axis=1, keepdims=True)
    fused = jnp.dot(ex, v_s[...], preferred_element_type=jnp.float32) / ssum
    logits = jnp.dot(fused, wo_ref[...], preferred_element_type=jnp.float32) + bo_ref[...]
    lm = jnp.max(logits, axis=1, keepdims=True)
    probs = jnp.exp(logits - lm)

    j = lax.broadcasted_iota(jnp.int32, (RB, NEXP), 1)
    m1 = jnp.max(probs, axis=1, keepdims=True)
    a1 = jnp.min(jnp.where(probs == m1, j, NEXP), axis=1, keepdims=True)
    masked = jnp.where(j == a1, -1.0, probs)
    m2 = jnp.max(masked, axis=1, keepdims=True)
    a2 = jnp.min(jnp.where(masked == m2, j, NEXP), axis=1, keepdims=True)
    den = m1 + m2 + 1e-9
    g1 = m1 / den
    g2 = m2 / den
    code = a1 * 16 + a2
    e2_ref[...] = jnp.where(j == 0, a1, jnp.where(j == 1, a2,
                            jnp.where(j == 2, code, 0)))
    g2_ref[...] = jnp.where(j == 0, g1, jnp.where(j == 1, g2, 0.0))


def _router(h, sf1, sf2, wqh, wqs, bq, wkh, wks, bk, wvh, wvs, bv, wo, bo):
    full = lambda shape: pl.BlockSpec(shape, lambda i: tuple(0 for _ in shape))
    return pl.pallas_call(
        _router_body,
        grid=(N // RB,),
        in_specs=[
            full((N, H)),
            full((N, 1)),
            full((N, 1)),
            full((H, H)), full((2, H)), full((1, H)),
            full((H, H)), full((2, H)), full((1, H)),
            full((H, H)), full((2, H)), full((1, H)),
            full((H, NEXP)), full((1, NEXP)),
        ],
        out_specs=[
            pl.BlockSpec((RB, NEXP), lambda i: (i, 0)),
            pl.BlockSpec((RB, NEXP), lambda i: (i, 0)),
        ],
        out_shape=[
            jax.ShapeDtypeStruct((N, NEXP), jnp.int32),
            jax.ShapeDtypeStruct((N, NEXP), jnp.float32),
        ],
        scratch_shapes=[
            pltpu.VMEM((N, H), jnp.float32),
            pltpu.VMEM((N, H), jnp.float32),
        ],
    )(h, sf1, sf2, wqh, wqs, bq, wkh, wks, bk, wvh, wvs, bv, wo, bo)


def _experts_body(h_ref, aggh_ref, we1_ref, be1_ref, we2_ref, be2_ref,
                  e2_ref, g2_ref, u_ref, self_ref, t_s):
    e = pl.program_id(0)

    @pl.when(e == 0)
    def _init():
        t_s[...] = h_ref[...] + aggh_ref[0] + aggh_ref[1]
        self_ref[...] = jnp.zeros((N, OUT), jnp.float32)

    he = jnp.dot(t_s[...], we1_ref[0], preferred_element_type=jnp.float32)
    he = jnp.maximum(he + be1_ref[0], 0.0)
    u = jnp.dot(he, we2_ref[0], preferred_element_type=jnp.float32)
    u_ref[0] = u
    coef = (jnp.where(e2_ref[:, 0:1] == e, g2_ref[:, 0:1], 0.0)
            + jnp.where(e2_ref[:, 1:2] == e, g2_ref[:, 1:2], 0.0))
    self_ref[...] += coef * (u + be2_ref[0])


def _experts(h, aggh, we1, be1, we2, be2, e2, g2):
    full = lambda shape: pl.BlockSpec(shape, lambda e: tuple(0 for _ in shape))
    return pl.pallas_call(
        _experts_body,
        grid=(NEXP,),
        in_specs=[
            full((N, H)),
            full((2, N, H)),
            pl.BlockSpec((1, H, H), lambda e: (e, 0, 0)),
            pl.BlockSpec((1, 1, H), lambda e: (e, 0, 0)),
            pl.BlockSpec((1, H, OUT), lambda e: (e, 0, 0)),
            pl.BlockSpec((1, 1, OUT), lambda e: (e, 0, 0)),
            full((N, NEXP)),
            full((N, NEXP)),
        ],
        out_specs=[
            pl.BlockSpec((1, N, OUT), lambda e: (e, 0, 0)),
            pl.BlockSpec((N, OUT), lambda e: (0, 0)),
        ],
        out_shape=[
            jax.ShapeDtypeStruct((NEXP, N, OUT), jnp.float32),
            jax.ShapeDtypeStruct((N, OUT), jnp.float32),
        ],
        scratch_shapes=[pltpu.VMEM((N, H), jnp.float32)],
    )(h, aggh, we1, be1, we2, be2, e2, g2)


EB = E // CH


def _edgeprep_body(code_ref, src_ref, dst_ref, i1_ref, i2_ref, db_ref):
    dstv = dst_ref[...]
    srcv = src_ref[...]
    hi = dstv // CH
    lo = dstv % CH
    code = jnp.zeros((EB, CH), jnp.int32)
    for r in range(N // CH):
        row = jnp.broadcast_to(code_ref[0:1, r * CH:(r + 1) * CH], (EB, CH))
        gr = jnp.take_along_axis(row, lo, axis=1)
        code = jnp.where(hi == r, gr, code)
    e1v = code // 16
    e2v = code % 16
    i1_ref[...] = e1v * N + srcv
    i2_ref[...] = e2v * N + srcv
    db_ref[...] = dstv + N


def _edgeprep(code_row, src2, dst2):
    return pl.pallas_call(
        _edgeprep_body,
        out_shape=[jax.ShapeDtypeStruct((EB, CH), jnp.int32)] * 3,
    )(code_row, src2, dst2)


def _sc_disp_body(u_hbm, idx1_hbm, idx2_hbm, dst_hbm, dst2_hbm, zer_hbm,
                  out_hbm, idx1_v, idx2_v, dst_v, dst2_v,
                  rows0_v, rows1_v, rows2_v,
                  acc_sh, sg0, sg1, sg2, ss0, ss1, ss2):
    c = lax.axis_index("c")
    s = lax.axis_index("s")
    w = c * NS + s
    rpt = 2 * N // NS
    for r in range(rpt // CH):
        pltpu.sync_copy(zer_hbm, acc_sh.at[pl.ds(s * rpt + r * CH, CH)])
    pltpu.sync_copy(idx1_hbm.at[w], idx1_v)
    pltpu.sync_copy(idx2_hbm.at[w], idx2_v)
    pltpu.sync_copy(dst_hbm.at[w], dst_v)
    pltpu.sync_copy(dst2_hbm.at[w], dst2_v)
    plsc.subcore_barrier()
    units = []
    for t in range(NCH):
        units.append((idx1_v, dst_v, t))
        units.append((idx2_v, dst2_v, t))
    nu = len(units)
    rows = (rows0_v, rows1_v, rows2_v)
    sg = (sg0, sg1, sg2)
    ss = (ss0, ss1, ss2)
    nb = 3
    g = [None] * nu
    sc = [None] * nu
    for k in range(nb):
        iv, dv, t = units[k]
        g[k] = pltpu.async_copy(u_hbm.at[iv.at[t]], rows[k], sg[k])
    for k in range(nu):
        b = k % nb
        iv, dv, t = units[k]
        g[k].wait()
        sc[k] = pltpu.async_copy(rows[b], acc_sh.at[dv.at[t]], ss[b], add=True)
        if k + nb < nu:
            sc[k].wait()
            iv2, dv2, t2 = units[k + nb]
            g[k + nb] = pltpu.async_copy(u_hbm.at[iv2.at[t2]], rows[b], sg[b])
    for k in range(nu - nb, nu):
        sc[k].wait()
    plsc.subcore_barrier()
    pltpu.sync_copy(acc_sh.at[pl.ds(s * rpt, rpt)],
                    out_hbm.at[pl.ds(c * 2 * N + s * rpt, rpt)])


def _sc_dispatch(u_flat, idx1, idx2, dst, dstb):
    zer = jnp.zeros((CH, OUT), jnp.float32)
    mesh = plsc.VectorSubcoreMesh(core_axis_name="c", subcore_axis_name="s")
    out = pl.kernel(
        _sc_disp_body,
        out_type=jax.ShapeDtypeStruct((NC * 2 * N, OUT), jnp.float32),
        mesh=mesh,
        scratch_types=[
            pltpu.VMEM((NCH, CH), jnp.int32),
            pltpu.VMEM((NCH, CH), jnp.int32),
            pltpu.VMEM((NCH, CH), jnp.int32),
            pltpu.VMEM((NCH, CH), jnp.int32),
            pltpu.VMEM((CH, OUT), jnp.float32),
            pltpu.VMEM((CH, OUT), jnp.float32),
            pltpu.VMEM((CH, OUT), jnp.float32),
            pltpu.VMEM_SHARED((2 * N, OUT), jnp.float32),
            pltpu.SemaphoreType.DMA,
            pltpu.SemaphoreType.DMA,
            pltpu.SemaphoreType.DMA,
            pltpu.SemaphoreType.DMA,
            pltpu.SemaphoreType.DMA,
            pltpu.SemaphoreType.DMA,
        ],
    )(u_flat,
      idx1.reshape(NW, NCH, CH), idx2.reshape(NW, NCH, CH),
      dst.reshape(NW, NCH, CH), dstb.reshape(NW, NCH, CH), zer)
    return out.reshape(NC, 2 * N, OUT)


def _combine_body(self_ref, acc_ref, g2_ref, o_ref):
    a1 = acc_ref[0, :N, :] + acc_ref[1, :N, :]
    a2 = acc_ref[0, N:, :] + acc_ref[1, N:, :]
    o_ref[...] = (self_ref[...]
                  + g2_ref[:, 0:1] * a1 + g2_ref[:, 1:2] * a2)


def _combine(selfp, acc, g2):
    return pl.pallas_call(
        _combine_body,
        out_shape=jax.ShapeDtypeStruct((N, OUT), jnp.float32),
    )(selfp, acc, g2)


def kernel(x, edge_index, batch, W_enc, b_enc, Wq, bq, Wk, bk, Wv, bv, Wo, bo,
           We1, be1, We2, be2):
    src = edge_index[0]
    dst = edge_index[1]
    w_pad = jnp.zeros((RAW, H), jnp.float32).at[4:10].set(W_enc)

    h = _encoder(x, w_pad, b_enc.reshape(1, H))
    aggh = _sc_agg(h, src, dst)
    sf1, sf2 = _sizefeat(dst, batch)

    e2g, g2g = _router(
        h, sf1.reshape(N, 1), sf2.reshape(N, 1),
        Wq[:H], Wq[H:], bq.reshape(1, H),
        Wk[:H], Wk[H:], bk.reshape(1, H),
        Wv[:H], Wv[H:], bv.reshape(1, H),
        Wo, bo.reshape(1, NEXP))

    u, selfp = _experts(h, aggh, We1, be1.reshape(NEXP, 1, H),
                        We2, be2.reshape(NEXP, 1, OUT), e2g, g2g)

    code_row = e2g[:, 2].reshape(1, N)
    eidx1, eidx2, edstb = _edgeprep(code_row, src.reshape(EB, CH),
                                    dst.reshape(EB, CH))
    acc = _sc_dispatch(u.reshape(NEXP * N, OUT), eidx1, eidx2, dst, edstb)
    return _combine(selfp, acc, g2g)


# speedup vs baseline: 1.8369x; 1.8369x over previous
import functools
import math

import jax
import jax.numpy as jnp
from jax import lax
from jax.experimental import pallas as pl
from jax.experimental.pallas import tpu as pltpu
from jax.experimental.pallas import tpu_sc as plsc

N = 4096
E = 65536
RAW = 16
H = 128
OUT = 128
NEXP = 8
NGRAPH = 8

NC = 2
NS = 16
NW = NC * NS
EW = E // NW
CH = 128
NCH = EW // CH
RB = 512


def _enc_body(x_ref, w_ref, b_ref, o_ref):
    h = jnp.dot(x_ref[...], w_ref[...], preferred_element_type=jnp.float32)
    o_ref[...] = jnp.maximum(h + b_ref[...], 0.0)


def _encoder(x, w_pad, b):
    return pl.pallas_call(
        _enc_body,
        out_shape=jax.ShapeDtypeStruct((N, H), jnp.float32),
    )(x, w_pad, b)


def _sc_agg_body(h_hbm, src_hbm, dst_hbm, zer_hbm, out_hbm,
                 src_v, dst_v, rows0_v, rows1_v, rows2_v,
                 acc_sh, sg0, sg1, sg2, ss0, ss1, ss2):
    c = lax.axis_index("c")
    s = lax.axis_index("s")
    w = c * NS + s
    rpt = N // NS
    for r in range(rpt // CH):
        pltpu.sync_copy(zer_hbm, acc_sh.at[pl.ds(s * rpt + r * CH, CH)])
    pltpu.sync_copy(src_hbm.at[w], src_v)
    pltpu.sync_copy(dst_hbm.at[w], dst_v)
    plsc.subcore_barrier()
    rows = (rows0_v, rows1_v, rows2_v)
    sg = (sg0, sg1, sg2)
    ss = (ss0, ss1, ss2)
    nb = 3
    g = [None] * NCH
    sc = [None] * NCH
    for k in range(nb):
        g[k] = pltpu.async_copy(h_hbm.at[src_v.at[k]], rows[k], sg[k])
    for k in range(NCH):
        b = k % nb
        g[k].wait()
        sc[k] = pltpu.async_copy(rows[b], acc_sh.at[dst_v.at[k]], ss[b], add=True)
        if k + nb < NCH:
            sc[k].wait()
            g[k + nb] = pltpu.async_copy(h_hbm.at[src_v.at[k + nb]], rows[b], sg[b])
    for k in range(NCH - nb, NCH):
        sc[k].wait()
    plsc.subcore_barrier()
    pltpu.sync_copy(acc_sh.at[pl.ds(s * rpt, rpt)],
                    out_hbm.at[pl.ds(c * N + s * rpt, rpt)])


def _sc_agg(h, src, dst):
    zer = jnp.zeros((CH, H), jnp.float32)
    src3 = src.reshape(NW, NCH, CH)
    dst3 = dst.reshape(NW, NCH, CH)
    mesh = plsc.VectorSubcoreMesh(core_axis_name="c", subcore_axis_name="s")
    out = pl.kernel(
        _sc_agg_body,
        out_type=jax.ShapeDtypeStruct((NC * N, H), jnp.float32),
        mesh=mesh,
        scratch_types=[
            pltpu.VMEM((NCH, CH), jnp.int32),
            pltpu.VMEM((NCH, CH), jnp.int32),
            pltpu.VMEM((CH, H), jnp.float32),
            pltpu.VMEM((CH, H), jnp.float32),
            pltpu.VMEM((CH, H), jnp.float32),
            pltpu.VMEM_SHARED((N, H), jnp.float32),
            pltpu.SemaphoreType.DMA,
            pltpu.SemaphoreType.DMA,
            pltpu.SemaphoreType.DMA,
            pltpu.SemaphoreType.DMA,
            pltpu.SemaphoreType.DMA,
            pltpu.SemaphoreType.DMA,
        ],
    )(h, src3, dst3, zer)
    return out.reshape(NC, N, H)


NHI = N // CH


DC = 8192


def _sizefeat_body(dr_ref, batch2_ref, sf1_ref, sf2_ref):
    hi_iota = lax.broadcasted_iota(jnp.int32, (NHI, DC), 0)
    lo_iota = lax.broadcasted_iota(jnp.int32, (CH, DC), 0)
    deg_t = jnp.zeros((CH, NHI), jnp.float32)
    for cix in range(E // DC):
        row = dr_ref[0:1, cix * DC:(cix + 1) * DC]
        a = (row // CH == hi_iota).astype(jnp.bfloat16)
        bt = (row % CH == lo_iota).astype(jnp.bfloat16)
        deg_t = deg_t + lax.dot_general(
            bt, a, (((1,), (1,)), ((), ())),
            preferred_element_type=jnp.float32)
    deg = jnp.transpose(deg_t)
    b2 = batch2_ref[...]
    gs = jnp.zeros((NHI, CH), jnp.float32)
    for g in range(NGRAPH):
        m = (b2 == g)
        cnt = jnp.sum(m.astype(jnp.float32))
        gs = gs + jnp.where(m, cnt, 0.0)
    sf1_ref[...] = jnp.log(1.0 + gs)
    sf2_ref[...] = jnp.log(1.0 + deg)


def _sizefeat(dst, batch):
    return pl.pallas_call(
        _sizefeat_body,
        out_shape=[jax.ShapeDtypeStruct((NHI, CH), jnp.float32)] * 2,
    )(dst.reshape(1, E), batch.reshape(NHI, CH))


def _router_body(h_ref, sf1_ref, sf2_ref,
                 wqh_ref, wqs_ref, bq_ref,
                 wkh_ref, wks_ref, bk_ref,
                 wvh_ref, wvs_ref, bv_ref,
                 wo_ref, bo_ref,
                 e2_ref, g2_ref,
                 k_s, v_s):
    i = pl.program_id(0)

    @pl.when(i == 0)
    def _init():
        sf1 = sf1_ref[...]
        sf2 = sf2_ref[...]
        h = h_ref[...]
        k_s[...] = (jnp.dot(h, wkh_ref[...], preferred_element_type=jnp.float32)
                    + sf1 * wks_ref[0:1, :] + sf2 * wks_ref[1:2, :] + bk_ref[...])
        v_s[...] = (jnp.dot(h, wvh_ref[...], preferred_element_type=jnp.float32)
                    + sf1 * wvs_ref[0:1, :] + sf2 * wvs_ref[1:2, :] + bv_ref[...])

    rows = pl.ds(i * RB, RB)
    hq = h_ref[rows, :]
    q = (jnp.dot(hq, wqh_ref[...], preferred_element_type=jnp.float32)
         + sf1_ref[rows, :] * wqs_ref[0:1, :]
         + sf2_ref[rows, :] * wqs_ref[1:2, :] + bq_ref[...])
    scores = lax.dot_general(q, k_s[...], (((1,), (1,)), ((), ())),
                             preferred_element_type=jnp.float32)
    ex = jnp.exp(scores / jnp.sqrt(jnp.float32(H + 2)))
    ssum = jnp.sum(ex, axis=1, keepdims=True)
    fused = jnp.dot(ex, v_s[...], preferred_element_type=jnp.float32) / ssum
    logits = jnp.dot(fused, wo_ref[...], preferred_element_type=jnp.float32) + bo_ref[...]
    lm = jnp.max(logits, axis=1, keepdims=True)
    probs = jnp.exp(logits - lm)

    j = lax.broadcasted_iota(jnp.int32, (RB, NEXP), 1)
    m1 = jnp.max(probs, axis=1, keepdims=True)
    a1 = jnp.min(jnp.where(probs == m1, j, NEXP), axis=1, keepdims=True)
    masked = jnp.where(j == a1, -1.0, probs)
    m2 = jnp.max(masked, axis=1, keepdims=True)
    a2 = jnp.min(jnp.where(masked == m2, j, NEXP), axis=1, keepdims=True)
    den = m1 + m2 + 1e-9
    g1 = m1 / den
    g2 = m2 / den
    code = a1 * 16 + a2
    e2_ref[...] = jnp.where(j == 0, a1, jnp.where(j == 1, a2,
                            jnp.where(j == 2, code, 0)))
    g2_ref[...] = jnp.where(j == 0, g1, jnp.where(j == 1, g2, 0.0))


def _router(h, sf1, sf2, wqh, wqs, bq, wkh, wks, bk, wvh, wvs, bv, wo, bo):
    full = lambda shape: pl.BlockSpec(shape, lambda i: tuple(0 for _ in shape))
    return pl.pallas_call(
        _router_body,
        grid=(N // RB,),
        in_specs=[
            full((N, H)),
            full((N, 1)),
            full((N, 1)),
            full((H, H)), full((2, H)), full((1, H)),
            full((H, H)), full((2, H)), full((1, H)),
            full((H, H)), full((2, H)), full((1, H)),
            full((H, NEXP)), full((1, NEXP)),
        ],
        out_specs=[
            pl.BlockSpec((RB, NEXP), lambda i: (i, 0)),
            pl.BlockSpec((RB, NEXP), lambda i: (i, 0)),
        ],
        out_shape=[
            jax.ShapeDtypeStruct((N, NEXP), jnp.int32),
            jax.ShapeDtypeStruct((N, NEXP), jnp.float32),
        ],
        scratch_shapes=[
            pltpu.VMEM((N, H), jnp.float32),
            pltpu.VMEM((N, H), jnp.float32),
        ],
    )(h, sf1, sf2, wqh, wqs, bq, wkh, wks, bk, wvh, wvs, bv, wo, bo)


def _experts_body(h_ref, aggh_ref, we1_ref, be1_ref, we2_ref, be2_ref,
                  e2_ref, g2_ref, u_ref, self_ref, t_s):
    e = pl.program_id(0)

    @pl.when(e == 0)
    def _init():
        t_s[...] = h_ref[...] + aggh_ref[0] + aggh_ref[1]
        self_ref[...] = jnp.zeros((N, OUT), jnp.float32)

    he = jnp.dot(t_s[...], we1_ref[0], preferred_element_type=jnp.float32)
    he = jnp.maximum(he + be1_ref[0], 0.0)
    u = jnp.dot(he, we2_ref[0], preferred_element_type=jnp.float32)
    u_ref[0] = u
    coef = (jnp.where(e2_ref[:, 0:1] == e, g2_ref[:, 0:1], 0.0)
            + jnp.where(e2_ref[:, 1:2] == e, g2_ref[:, 1:2], 0.0))
    self_ref[...] += coef * (u + be2_ref[0])


def _experts(h, aggh, we1, be1, we2, be2, e2, g2):
    full = lambda shape: pl.BlockSpec(shape, lambda e: tuple(0 for _ in shape))
    return pl.pallas_call(
        _experts_body,
        grid=(NEXP,),
        in_specs=[
            full((N, H)),
            full((2, N, H)),
            pl.BlockSpec((1, H, H), lambda e: (e, 0, 0)),
            pl.BlockSpec((1, 1, H), lambda e: (e, 0, 0)),
            pl.BlockSpec((1, H, OUT), lambda e: (e, 0, 0)),
            pl.BlockSpec((1, 1, OUT), lambda e: (e, 0, 0)),
            full((N, NEXP)),
            full((N, NEXP)),
        ],
        out_specs=[
            pl.BlockSpec((1, N, OUT), lambda e: (e, 0, 0)),
            pl.BlockSpec((N, OUT), lambda e: (0, 0)),
        ],
        out_shape=[
            jax.ShapeDtypeStruct((NEXP, N, OUT), jnp.float32),
            jax.ShapeDtypeStruct((N, OUT), jnp.float32),
        ],
        scratch_shapes=[pltpu.VMEM((N, H), jnp.float32)],
    )(h, aggh, we1, be1, we2, be2, e2, g2)


EB = E // CH


def _edgeprep_body(code_ref, src_ref, dst_ref, i1_ref, i2_ref, db_ref):
    dstv = dst_ref[...]
    srcv = src_ref[...]
    hi = dstv // CH
    lo = dstv % CH
    code = jnp.zeros((EB, CH), jnp.int32)
    for r in range(N // CH):
        row = jnp.broadcast_to(code_ref[0:1, r * CH:(r + 1) * CH], (EB, CH))
        gr = jnp.take_along_axis(row, lo, axis=1)
        code = jnp.where(hi == r, gr, code)
    e1v = code // 16
    e2v = code % 16
    i1_ref[...] = e1v * N + srcv
    i2_ref[...] = e2v * N + srcv
    db_ref[...] = dstv + N


def _edgeprep(code_row, src2, dst2):
    return pl.pallas_call(
        _edgeprep_body,
        out_shape=[jax.ShapeDtypeStruct((EB, CH), jnp.int32)] * 3,
    )(code_row, src2, dst2)


def _sc_disp_body(u_hbm, idx1_hbm, idx2_hbm, dst_hbm, dst2_hbm, zer_hbm,
                  out_hbm, idx1_v, idx2_v, dst_v, dst2_v,
                  rows0_v, rows1_v, rows2_v,
                  acc_sh, sg0, sg1, sg2, ss0, ss1, ss2):
    c = lax.axis_index("c")
    s = lax.axis_index("s")
    w = c * NS + s
    rpt = 2 * N // NS
    for r in range(rpt // CH):
        pltpu.sync_copy(zer_hbm, acc_sh.at[pl.ds(s * rpt + r * CH, CH)])
    pltpu.sync_copy(idx1_hbm.at[w], idx1_v)
    pltpu.sync_copy(idx2_hbm.at[w], idx2_v)
    pltpu.sync_copy(dst_hbm.at[w], dst_v)
    pltpu.sync_copy(dst2_hbm.at[w], dst2_v)
    plsc.subcore_barrier()
    units = []
    for t in range(NCH):
        units.append((idx1_v, dst_v, t))
        units.append((idx2_v, dst2_v, t))
    nu = len(units)
    rows = (rows0_v, rows1_v, rows2_v)
    sg = (sg0, sg1, sg2)
    ss = (ss0, ss1, ss2)
    nb = 3
    g = [None] * nu
    sc = [None] * nu
    for k in range(nb):
        iv, dv, t = units[k]
        g[k] = pltpu.async_copy(u_hbm.at[iv.at[t]], rows[k], sg[k])
    for k in range(nu):
        b = k % nb
        iv, dv, t = units[k]
        g[k].wait()
        sc[k] = pltpu.async_copy(rows[b], acc_sh.at[dv.at[t]], ss[b], add=True)
        if k + nb < nu:
            sc[k].wait()
            iv2, dv2, t2 = units[k + nb]
            g[k + nb] = pltpu.async_copy(u_hbm.at[iv2.at[t2]], rows[b], sg[b])
    for k in range(nu - nb, nu):
        sc[k].wait()
    plsc.subcore_barrier()
    pltpu.sync_copy(acc_sh.at[pl.ds(s * rpt, rpt)],
                    out_hbm.at[pl.ds(c * 2 * N + s * rpt, rpt)])


def _sc_dispatch(u_flat, idx1, idx2, dst, dstb):
    zer = jnp.zeros((CH, OUT), jnp.float32)
    mesh = plsc.VectorSubcoreMesh(core_axis_name="c", subcore_axis_name="s")
    out = pl.kernel(
        _sc_disp_body,
        out_type=jax.ShapeDtypeStruct((NC * 2 * N, OUT), jnp.float32),
        mesh=mesh,
        scratch_types=[
            pltpu.VMEM((NCH, CH), jnp.int32),
            pltpu.VMEM((NCH, CH), jnp.int32),
            pltpu.VMEM((NCH, CH), jnp.int32),
            pltpu.VMEM((NCH, CH), jnp.int32),
            pltpu.VMEM((CH, OUT), jnp.float32),
            pltpu.VMEM((CH, OUT), jnp.float32),
            pltpu.VMEM((CH, OUT), jnp.float32),
            pltpu.VMEM_SHARED((2 * N, OUT), jnp.float32),
            pltpu.SemaphoreType.DMA,
            pltpu.SemaphoreType.DMA,
            pltpu.SemaphoreType.DMA,
            pltpu.SemaphoreType.DMA,
            pltpu.SemaphoreType.DMA,
            pltpu.SemaphoreType.DMA,
        ],
    )(u_flat,
      idx1.reshape(NW, NCH, CH), idx2.reshape(NW, NCH, CH),
      dst.reshape(NW, NCH, CH), dstb.reshape(NW, NCH, CH), zer)
    return out.reshape(NC, 2 * N, OUT)


def _combine_body(self_ref, acc_ref, g2_ref, o_ref):
    a1 = acc_ref[0, :N, :] + acc_ref[1, :N, :]
    a2 = acc_ref[0, N:, :] + acc_ref[1, N:, :]
    o_ref[...] = (self_ref[...]
                  + g2_ref[:, 0:1] * a1 + g2_ref[:, 1:2] * a2)


def _combine(selfp, acc, g2):
    return pl.pallas_call(
        _combine_body,
        out_shape=jax.ShapeDtypeStruct((N, OUT), jnp.float32),
    )(selfp, acc, g2)


def kernel(x, edge_index, batch, W_enc, b_enc, Wq, bq, Wk, bk, Wv, bv, Wo, bo,
           We1, be1, We2, be2):
    src = edge_index[0]
    dst = edge_index[1]
    w_pad = jnp.zeros((RAW, H), jnp.float32).at[4:10].set(W_enc)

    h = _encoder(x, w_pad, b_enc.reshape(1, H))
    aggh = _sc_agg(h, src, dst)
    sf1, sf2 = _sizefeat(dst, batch)

    e2g, g2g = _router(
        h, sf1.reshape(N, 1), sf2.reshape(N, 1),
        Wq[:H], Wq[H:], bq.reshape(1, H),
        Wk[:H], Wk[H:], bk.reshape(1, H),
        Wv[:H], Wv[H:], bv.reshape(1, H),
        Wo, bo.reshape(1, NEXP))

    u, selfp = _experts(h, aggh, We1, be1.reshape(NEXP, 1, H),
                        We2, be2.reshape(NEXP, 1, OUT), e2g, g2g)

    code_row = e2g[:, 2].reshape(1, N)
    eidx1, eidx2, edstb = _edgeprep(code_row, src.reshape(EB, CH),
                                    dst.reshape(EB, CH))
    acc = _sc_dispatch(u.reshape(NEXP * N, OUT), eidx1, eidx2, dst, edstb)
    return _combine(selfp, acc, g2g)
